# baseline (speedup 1.0000x reference)
import jax
import jax.numpy as jnp
from jax import lax
from jax.experimental import pallas as pl
from jax.experimental.pallas import tpu as pltpu
from jax.experimental.pallas import tpu_sc as plsc

NROW = 128
NCOL = 100000
NSAMP = 256
L = 16
NB = NCOL // L
SEG = 3120
SEGB = SEG // L
HALF = SEG * L
HBLK = HALF // L
TAIL = NCOL - 2 * HALF
TAILB = TAIL // L
G16W = 6256


def _iota16():
    return lax.iota(jnp.int32, 16)


def _bcast_i32(x):
    return x + jnp.zeros((16,), jnp.int32)


def _body(dist_hbm, u_hbm, out_hbm,
          buf, g16, uv, outbuf, s16,
          sem_a, sem_b, sem_t):
    cid = lax.axis_index("c")
    sid = lax.axis_index("s")
    wid = cid * 16 + sid
    iota = _iota16()
    blk_iota = iota * SEGB
    zeros_f = jnp.zeros((16,), jnp.float32)
    zeros_i = jnp.zeros((16,), jnp.int32)
    ones_i = jnp.ones((16,), jnp.int32)

    pltpu.sync_copy(u_hbm.at[pl.ds(wid * 4 * NSAMP, 4 * NSAMP)], uv)

    def compute_half(eoff, blkoff):
        init = (zeros_f, iota * SEG + eoff, blk_iota + blkoff)

        def blk_body(b, carry):
            acc, idxv, colv = carry
            vals = [plsc.load_gather(buf, [idxv + j]) for j in range(L)]
            while len(vals) > 1:
                vals = [vals[k] + vals[k + 1] for k in range(0, len(vals), 2)]
            acc = acc + vals[0]
            plsc.store_scatter(g16, [colv], acc)
            return acc, idxv + L, colv + 1

        acc, _, _ = plsc.parallel_loop(0, SEGB, carry=init,
                                       unroll=5)(blk_body)
        return acc

    def fixup_half(blkoff, base):
        def fix_body(b):
            idx = blk_iota + (blkoff + b)
            v = plsc.load_gather(g16, [idx])
            plsc.store_scatter(g16, [idx], v + base)

        plsc.parallel_loop(0, SEGB)(fix_body)

    def search16(t):
        p = zeros_i
        for s in (4096, 2048, 1024, 512, 256, 128, 64, 32, 16, 8, 4, 2, 1):
            cand = p + s
            col = jnp.minimum(cand - 1, G16W - 1)
            val = plsc.load_gather(g16, [col])
            ok = jnp.logical_and(cand <= NB, val <= t)
            p = jnp.where(ok, cand, p)
        base = jnp.where(
            p > 0, plsc.load_gather(g16, [jnp.maximum(p - 1, 0)]), zeros_f)
        thr = t - base
        e = jnp.minimum(p, NB - 1) * L
        run = zeros_f
        cnt = zeros_i
        for j in range(L):
            val = plsc.load_gather(buf, [jnp.minimum(e + j, NCOL - 1)])
            run = run + val
            cnt = cnt + jnp.where(run <= thr, ones_i, zeros_i)
        return jnp.minimum(jnp.maximum(p * L + cnt, 0), NCOL - 1)

    def row_body(rl, _):
        row = wid * 4 + rl
        cp_a = pltpu.make_async_copy(
            dist_hbm.at[row, pl.ds(0, HALF)], buf.at[pl.ds(0, HALF)], sem_a)
        cp_b = pltpu.make_async_copy(
            dist_hbm.at[row, pl.ds(HALF, HALF)],
            buf.at[pl.ds(HALF, HALF)], sem_b)
        cp_t = pltpu.make_async_copy(
            dist_hbm.at[row, pl.ds(2 * HALF, TAIL)],
            buf.at[pl.ds(2 * HALF, TAIL)], sem_t)
        cp_a.start()
        cp_b.start()
        cp_t.start()

        cp_a.wait()
        acc_a = compute_half(0, 0)
        cp_b.wait()
        acc_b = compute_half(HALF, HBLK)

        cum_a = plsc.cumsum(acc_a)
        base_a = cum_a - acc_a
        s16[...] = cum_a
        tot_a = plsc.load_gather(s16, [jnp.full((16,), 15, jnp.int32)])
        cum_b = plsc.cumsum(acc_b)
        base_b = cum_b - acc_b + tot_a
        s16[...] = cum_b + tot_a
        tot_ab = plsc.load_gather(s16, [jnp.full((16,), 15, jnp.int32)])
        fixup_half(0, base_a)
        fixup_half(HBLK, base_b)

        cp_t.wait()
        lane0 = iota == 0
        tcum = tot_ab
        for t in range(TAILB):
            tcum = tcum + jnp.sum(buf[pl.ds(2 * HALF + t * L, L)])
            plsc.store_scatter(g16, [_bcast_i32(2 * HBLK + t)], tcum,
                               mask=lane0)

        tot = plsc.load_gather(g16, [jnp.full((16,), NB - 1, jnp.int32)])

        def samp_body(jv, _):
            o1 = rl * NSAMP + jv * 16
            o2 = o1 + 128
            idx1 = search16(uv[pl.ds(o1, 16)] * tot)
            idx2 = search16(uv[pl.ds(o2, 16)] * tot)
            outbuf[pl.ds(o1, 16)] = idx1
            outbuf[pl.ds(o2, 16)] = idx2
            return 0

        lax.fori_loop(0, 8, samp_body, 0)
        return 0

    lax.fori_loop(0, 4, row_body, 0)
    pltpu.sync_copy(outbuf, out_hbm.at[pl.ds(wid * 4 * NSAMP, 4 * NSAMP)])


def kernel(dist):
    mesh = plsc.VectorSubcoreMesh(core_axis_name="c", subcore_axis_name="s")
    params = pltpu.CompilerParams(use_tc_tiling_on_sc=False,
                                  needs_layout_passes=False)

    ukey = jax.random.fold_in(jax.random.key(0), 1)
    u = jax.random.uniform(ukey, (NROW, NSAMP), dtype=jnp.float32)

    run = pl.kernel(
        _body,
        out_type=jax.ShapeDtypeStruct((NROW * NSAMP,), jnp.int32),
        mesh=mesh,
        compiler_params=params,
        scratch_types=[
            pltpu.VMEM((NCOL,), jnp.float32),
            pltpu.VMEM((G16W,), jnp.float32),
            pltpu.VMEM((4 * NSAMP,), jnp.float32),
            pltpu.VMEM((4 * NSAMP,), jnp.int32),
            pltpu.VMEM((16,), jnp.float32),
            pltpu.SemaphoreType.DMA,
            pltpu.SemaphoreType.DMA,
            pltpu.SemaphoreType.DMA,
        ],
    )
    return run(dist, u.reshape(NROW * NSAMP)).reshape(NROW, NSAMP)


# speedup vs baseline: 1.0350x; 1.0350x over previous
import jax
import jax.numpy as jnp
from jax import lax
from jax.experimental import pallas as pl
from jax.experimental.pallas import tpu as pltpu
from jax.experimental.pallas import tpu_sc as plsc

NROW = 128
NCOL = 100000
NSAMP = 256
L = 16
NB = NCOL // L
SEG = 3120
SEGB = SEG // L
HALF = SEG * L
HBLK = HALF // L
TAIL = NCOL - 2 * HALF
TAILB = TAIL // L
G16W = 6256


def _iota16():
    return lax.iota(jnp.int32, 16)


def _bcast_i32(x):
    return x + jnp.zeros((16,), jnp.int32)


def _body(dist_hbm, u_hbm, out_hbm,
          buf, g16, uv, outbuf, s16,
          sem_a, sem_b, sem_t):
    cid = lax.axis_index("c")
    sid = lax.axis_index("s")
    wid = cid * 16 + sid
    iota = _iota16()
    blk_iota = iota * SEGB
    zeros_f = jnp.zeros((16,), jnp.float32)
    zeros_i = jnp.zeros((16,), jnp.int32)
    ones_i = jnp.ones((16,), jnp.int32)

    pltpu.sync_copy(u_hbm.at[pl.ds(wid * 4 * NSAMP, 4 * NSAMP)], uv)

    def compute_half(eoff, blkoff):
        init = (zeros_f, iota * SEG + eoff, blk_iota + blkoff)

        def blk_body(b, carry):
            acc, idxv, colv = carry
            vals = [plsc.load_gather(buf, [idxv + j]) for j in range(L)]
            while len(vals) > 1:
                vals = [vals[k] + vals[k + 1] for k in range(0, len(vals), 2)]
            acc = acc + vals[0]
            plsc.store_scatter(g16, [colv], acc)
            return acc, idxv + L, colv + 1

        acc, _, _ = plsc.parallel_loop(0, SEGB, carry=init)(blk_body)
        return acc

    def fixup_half(blkoff, base):
        def fix_body(b):
            idx = blk_iota + (blkoff + b)
            v = plsc.load_gather(g16, [idx])
            plsc.store_scatter(g16, [idx], v + base)

        plsc.parallel_loop(0, SEGB)(fix_body)

    def search16(t):
        p = zeros_i
        for s in (4096, 2048, 1024, 512, 256, 128, 64, 32, 16, 8, 4, 2, 1):
            cand = p + s
            col = jnp.minimum(cand - 1, G16W - 1)
            val = plsc.load_gather(g16, [col])
            ok = jnp.logical_and(cand <= NB, val <= t)
            p = jnp.where(ok, cand, p)
        base = jnp.where(
            p > 0, plsc.load_gather(g16, [jnp.maximum(p - 1, 0)]), zeros_f)
        thr = t - base
        e = jnp.minimum(p, NB - 1) * L
        run = zeros_f
        cnt = zeros_i
        for j in range(L):
            val = plsc.load_gather(buf, [jnp.minimum(e + j, NCOL - 1)])
            run = run + val
            cnt = cnt + jnp.where(run <= thr, ones_i, zeros_i)
        return jnp.minimum(jnp.maximum(p * L + cnt, 0), NCOL - 1)

    def row_body(rl, _):
        row = wid * 4 + rl
        cp_a = pltpu.make_async_copy(
            dist_hbm.at[row, pl.ds(0, HALF)], buf.at[pl.ds(0, HALF)], sem_a)
        cp_b = pltpu.make_async_copy(
            dist_hbm.at[row, pl.ds(HALF, HALF)],
            buf.at[pl.ds(HALF, HALF)], sem_b)
        cp_t = pltpu.make_async_copy(
            dist_hbm.at[row, pl.ds(2 * HALF, TAIL)],
            buf.at[pl.ds(2 * HALF, TAIL)], sem_t)
        cp_a.start()
        cp_b.start()
        cp_t.start()

        cp_a.wait()
        acc_a = compute_half(0, 0)
        cp_b.wait()
        acc_b = compute_half(HALF, HBLK)

        cum_a = plsc.cumsum(acc_a)
        base_a = cum_a - acc_a
        s16[...] = cum_a
        tot_a = plsc.load_gather(s16, [jnp.full((16,), 15, jnp.int32)])
        cum_b = plsc.cumsum(acc_b)
        base_b = cum_b - acc_b + tot_a
        s16[...] = cum_b + tot_a
        tot_ab = plsc.load_gather(s16, [jnp.full((16,), 15, jnp.int32)])
        fixup_half(0, base_a)
        fixup_half(HBLK, base_b)

        cp_t.wait()
        lane0 = iota == 0
        tcum = tot_ab
        for t in range(TAILB):
            tcum = tcum + jnp.sum(buf[pl.ds(2 * HALF + t * L, L)])
            plsc.store_scatter(g16, [_bcast_i32(2 * HBLK + t)], tcum,
                               mask=lane0)

        tot = plsc.load_gather(g16, [jnp.full((16,), NB - 1, jnp.int32)])

        def samp_body(jv, _):
            o1 = rl * NSAMP + jv * 16
            o2 = o1 + 128
            idx1 = search16(uv[pl.ds(o1, 16)] * tot)
            idx2 = search16(uv[pl.ds(o2, 16)] * tot)
            outbuf[pl.ds(o1, 16)] = idx1
            outbuf[pl.ds(o2, 16)] = idx2
            return 0

        lax.fori_loop(0, 8, samp_body, 0)
        return 0

    lax.fori_loop(0, 4, row_body, 0)
    pltpu.sync_copy(outbuf, out_hbm.at[pl.ds(wid * 4 * NSAMP, 4 * NSAMP)])


def kernel(dist):
    mesh = plsc.VectorSubcoreMesh(core_axis_name="c", subcore_axis_name="s")
    params = pltpu.CompilerParams(use_tc_tiling_on_sc=False,
                                  needs_layout_passes=False)

    ukey = jax.random.fold_in(jax.random.key(0), 1)
    u = jax.random.uniform(ukey, (NROW, NSAMP), dtype=jnp.float32)

    run = pl.kernel(
        _body,
        out_type=jax.ShapeDtypeStruct((NROW * NSAMP,), jnp.int32),
        mesh=mesh,
        compiler_params=params,
        scratch_types=[
            pltpu.VMEM((NCOL,), jnp.float32),
            pltpu.VMEM((G16W,), jnp.float32),
            pltpu.VMEM((4 * NSAMP,), jnp.float32),
            pltpu.VMEM((4 * NSAMP,), jnp.int32),
            pltpu.VMEM((16,), jnp.float32),
            pltpu.SemaphoreType.DMA,
            pltpu.SemaphoreType.DMA,
            pltpu.SemaphoreType.DMA,
        ],
    )
    return run(dist, u.reshape(NROW * NSAMP)).reshape(NROW, NSAMP)
